# baseline (speedup 1.0000x reference)
import functools

import jax
import jax.numpy as jnp
from jax.experimental import pallas as pl
from jax.experimental.pallas import tpu as pltpu


_CFG = [
    (64, True), (64, True), 'M',
    (128, True), (128, True), 'M',
    (256, True), (256, True), (256, True), (256, True), 'M',
    (512, True), (512, True), (512, True), (512, True), 'M',
    (512, True), (512, True), (512, True), (512, False),
]


def _layers():
    out, cin, i = [], 3, 0
    while i < len(_CFG):
        cout, relu = _CFG[i]
        pool = (i + 1 < len(_CFG)) and _CFG[i + 1] == 'M'
        out.append(dict(cin=cin, cout=cout, relu=relu, pool=pool))
        cin = cout
        i += 2 if pool else 1
    return out


_LAYERS = _layers()
_GROUPS = [[0, 1], [2, 3], [4, 5, 6, 7], [8, 9, 10, 11], [12, 13, 14, 15]]


def _first_conv_planar(x3, w27, b64):
    C, L = x3.shape
    W = int(round(L ** 0.5))
    H = L // W
    lane = jax.lax.broadcasted_iota(jnp.int32, (C, L), 1) % W
    z1 = jnp.zeros((C, 1), jnp.bfloat16)
    sr = jnp.concatenate([z1, x3[:, :L - 1]], axis=1)
    sr = sr * (lane != 0).astype(jnp.bfloat16)
    sl = jnp.concatenate([x3[:, 1:], z1], axis=1)
    sl = sl * (lane != W - 1).astype(jnp.bfloat16)
    dx_rows = [sr, x3, sl]
    zW = jnp.zeros((C, W), jnp.bfloat16)
    rows = []
    for ky in range(3):
        for u in dx_rows:
            if ky == 0:
                rows.append(jnp.concatenate([zW, u[:, :L - W]], axis=1))
            elif ky == 2:
                rows.append(jnp.concatenate([u[:, W:], zW], axis=1))
            else:
                rows.append(u)
    taps = jnp.concatenate(rows, axis=0)
    acc = jnp.dot(w27, taps, preferred_element_type=jnp.float32)
    acc = jnp.maximum(acc + b64, 0.0)
    y = jnp.transpose(acc.astype(jnp.bfloat16))
    return y.reshape(H, W, 64)


def _conv_bias_act(x, w3, b2, *, relu, pool):
    H, W, Cin = x.shape
    Cout = w3.shape[-1]

    zcol = jnp.zeros((H, 1, Cin), jnp.bfloat16)
    x_l = jnp.concatenate([zcol, x[:, :W - 1, :]], axis=1)
    x_r = jnp.concatenate([x[:, 1:, :], zcol], axis=1)
    xc = jnp.concatenate([x_l, x, x_r], axis=-1)
    zrow = jnp.zeros((1, W, 3 * Cin), jnp.bfloat16)
    xcp = jnp.concatenate([zrow, xc, zrow], axis=0)

    if w3.ndim == 2:
        xs = jnp.concatenate(
            [xcp[dy:dy + H] for dy in range(3)], axis=-1)
        acc = jnp.dot(xs.reshape(H * W, 9 * Cin), w3,
                      preferred_element_type=jnp.float32)
    else:
        acc = jnp.zeros((H * W, Cout), jnp.float32)
        for dy in range(3):
            xs = xcp[dy:dy + H].reshape(H * W, 3 * Cin)
            acc = acc + jnp.dot(xs, w3[dy],
                                preferred_element_type=jnp.float32)

    acc = acc + b2
    if relu:
        acc = jnp.maximum(acc, 0.0)

    if pool:
        ho, wo = H // 2, W // 2
        z = acc.reshape(H * wo, 2, Cout)
        z = jnp.maximum(z[:, 0, :], z[:, 1, :])
        z = z.reshape(ho, 2, wo, Cout)
        z = jnp.maximum(z[:, 0], z[:, 1])
        return z.astype(jnp.bfloat16)
    return acc.reshape(H, W, Cout).astype(jnp.bfloat16)


def _group_kernel(*refs, layers, first_planar, batch):
    x_ref = refs[0]
    o_ref = refs[-1]
    for b in range(batch):
        start = 0
        if first_planar:
            x = _first_conv_planar(x_ref[b].astype(jnp.bfloat16),
                                   refs[1][...], refs[2][...])
            start = 1
        else:
            x = x_ref[b]
        for i, lay in enumerate(layers[start:], start=start):
            w_ref = refs[1 + 2 * i]
            b_ref = refs[2 + 2 * i]
            x = _conv_bias_act(x, w_ref[...], b_ref[...],
                               relu=lay['relu'], pool=lay['pool'])
        o_ref[b] = x


def _run_group(x, params, layers, first_planar=False, batch=1):
    if first_planar:
        N = x.shape[0]
        H = W = int(round(x.shape[2] ** 0.5))
    else:
        N, H, W, _ = x.shape
    last = layers[-1]
    Ho, Wo = (H // 2, W // 2) if last['pool'] else (H, W)
    Cout = last['cout']

    in_specs = [pl.BlockSpec(
        (batch,) + x.shape[1:], lambda n: (n,) + (0,) * (x.ndim - 1))]
    args = [x]
    for (w, b), lay in zip(params, layers):
        in_specs.append(pl.BlockSpec(w.shape, lambda n, _nd=w.ndim: (0,) * _nd))
        in_specs.append(pl.BlockSpec(b.shape, lambda n, _nd=b.ndim: (0,) * _nd))
        args.append(w)
        args.append(b)

    kern = functools.partial(_group_kernel, layers=layers,
                             first_planar=first_planar, batch=batch)
    return pl.pallas_call(
        kern,
        out_shape=jax.ShapeDtypeStruct((N, Ho, Wo, Cout), jnp.bfloat16),
        grid=(N // batch,),
        in_specs=in_specs,
        out_specs=pl.BlockSpec((batch, Ho, Wo, Cout), lambda n: (n, 0, 0, 0)),
        compiler_params=pltpu.CompilerParams(
            dimension_semantics=("arbitrary",),
            vmem_limit_bytes=96 << 20,
        ),
    )(*args)


def kernel(img, w0, b0, w1, b1, w2, b2, w3, b3, w4, b4, w5, b5, w6, b6,
           w7, b7, w8, b8, w9, b9, w10, b10, w11, b11, w12, b12, w13, b13,
           w14, b14, w15, b15):
    ws = [w0, w1, w2, w3, w4, w5, w6, w7, w8, w9, w10, w11, w12, w13, w14, w15]
    bs = [b0, b1, b2, b3, b4, b5, b6, b7, b8, b9, b10, b11, b12, b13, b14, b15]
    params = []
    for i, (w, b, lay) in enumerate(zip(ws, bs, _LAYERS)):
        if i == 0:
            params.append((jnp.transpose(w.reshape(27, lay['cout'])),
                           b.reshape(lay['cout'], 1)))
        elif (3 * lay['cin']) % 128 == 0:
            params.append((w.reshape(9 * lay['cin'], lay['cout']),
                           b.reshape(1, lay['cout'])))
        else:
            params.append((w.reshape(3, 3 * lay['cin'], lay['cout']),
                           b.reshape(1, lay['cout'])))

    x = img.reshape(img.shape[0], 3, -1)
    for gi, g in enumerate(_GROUPS):
        batch = 2 if gi >= 2 and img.shape[0] % 2 == 0 else 1
        x = _run_group(x, [params[i] for i in g], [_LAYERS[i] for i in g],
                       first_planar=(gi == 0), batch=batch)
    return jnp.transpose(x, (0, 3, 1, 2)).astype(jnp.float32)


# speedup vs baseline: 1.0440x; 1.0440x over previous
import functools

import jax
import jax.numpy as jnp
from jax.experimental import pallas as pl
from jax.experimental.pallas import tpu as pltpu


_CFG = [
    (64, True), (64, True), 'M',
    (128, True), (128, True), 'M',
    (256, True), (256, True), (256, True), (256, True), 'M',
    (512, True), (512, True), (512, True), (512, True), 'M',
    (512, True), (512, True), (512, True), (512, False),
]


def _layers():
    out, cin, i = [], 3, 0
    while i < len(_CFG):
        cout, relu = _CFG[i]
        pool = (i + 1 < len(_CFG)) and _CFG[i + 1] == 'M'
        out.append(dict(cin=cin, cout=cout, relu=relu, pool=pool))
        cin = cout
        i += 2 if pool else 1
    return out


_LAYERS = _layers()


def _first_conv_planar(x3, w27, b64):
    C, L = x3.shape
    W = int(round(L ** 0.5))
    lane = jax.lax.broadcasted_iota(jnp.int32, (C, L), 1) % W
    z1 = jnp.zeros((C, 1), jnp.bfloat16)
    sr = jnp.concatenate([z1, x3[:, :L - 1]], axis=1)
    sr = sr * (lane != 0).astype(jnp.bfloat16)
    sl = jnp.concatenate([x3[:, 1:], z1], axis=1)
    sl = sl * (lane != W - 1).astype(jnp.bfloat16)
    dx_rows = [sr, x3, sl]
    zW = jnp.zeros((C, W), jnp.bfloat16)
    rows = []
    for ky in range(3):
        for u in dx_rows:
            if ky == 0:
                rows.append(jnp.concatenate([zW, u[:, :L - W]], axis=1))
            elif ky == 2:
                rows.append(jnp.concatenate([u[:, W:], zW], axis=1))
            else:
                rows.append(u)
    taps = jnp.concatenate(rows, axis=0)
    acc = jnp.dot(w27, taps, preferred_element_type=jnp.float32)
    acc = jnp.maximum(acc + b64, 0.0)
    return acc.astype(jnp.bfloat16)


def _conv_bias_act(x, w3, b2, *, relu, pool):
    H, W, Cin = x.shape
    Cout = w3.shape[-1]

    zcol = jnp.zeros((H, 1, Cin), jnp.bfloat16)
    x_l = jnp.concatenate([zcol, x[:, :W - 1, :]], axis=1)
    x_r = jnp.concatenate([x[:, 1:, :], zcol], axis=1)
    xc = jnp.concatenate([x_l, x, x_r], axis=-1)
    zrow = jnp.zeros((1, W, 3 * Cin), jnp.bfloat16)
    xcp = jnp.concatenate([zrow, xc, zrow], axis=0)

    if w3.ndim == 2:
        xs = jnp.concatenate(
            [xcp[dy:dy + H] for dy in range(3)], axis=-1)
        acc = jnp.dot(xs.reshape(H * W, 9 * Cin), w3,
                      preferred_element_type=jnp.float32)
    else:
        acc = jnp.zeros((H * W, Cout), jnp.float32)
        for dy in range(3):
            xs = xcp[dy:dy + H].reshape(H * W, 3 * Cin)
            acc = acc + jnp.dot(xs, w3[dy],
                                preferred_element_type=jnp.float32)

    acc = acc + b2
    if relu:
        acc = jnp.maximum(acc, 0.0)

    if pool:
        ho, wo = H // 2, W // 2
        z = acc.reshape(H * wo, 2, Cout)
        z = jnp.maximum(z[:, 0, :], z[:, 1, :])
        z = z.reshape(ho, 2, wo, Cout)
        z = jnp.maximum(z[:, 0], z[:, 1])
        return z.astype(jnp.bfloat16)
    return acc.reshape(H, W, Cout).astype(jnp.bfloat16)


def _stage_a_kernel(x_ref, w27_ref, b1_ref, w2_ref, b2_ref, o_ref):
    y0 = _first_conv_planar(x_ref[0].astype(jnp.bfloat16),
                            w27_ref[...], b1_ref[...])
    y1 = _first_conv_planar(x_ref[1].astype(jnp.bfloat16),
                            w27_ref[...], b1_ref[...])
    yt = jnp.concatenate([y0, y1], axis=0)
    L = yt.shape[1]
    W = int(round(L ** 0.5))
    y = jnp.transpose(yt).reshape(L // W, W, 128)
    o_ref[0] = _conv_bias_act(y, w2_ref[...], b2_ref[...],
                              relu=True, pool=True)


def _group_kernel(*refs, layers, batch, split_lanes):
    x_ref = refs[0]
    o_ref = refs[-1]
    for b in range(batch):
        if split_lanes:
            cin = layers[0]['cin']
            x = x_ref[0][..., b * cin:(b + 1) * cin]
        else:
            x = x_ref[b]
        for i, lay in enumerate(layers):
            x = _conv_bias_act(x, refs[1 + 2 * i][...], refs[2 + 2 * i][...],
                               relu=lay['relu'], pool=lay['pool'])
        o_ref[b] = x


def _const_specs(params):
    specs, args = [], []
    for w, b in params:
        specs.append(pl.BlockSpec(w.shape, lambda n, _nd=w.ndim: (0,) * _nd))
        specs.append(pl.BlockSpec(b.shape, lambda n, _nd=b.ndim: (0,) * _nd))
        args.append(w)
        args.append(b)
    return specs, args


def _compiler_params():
    return pltpu.CompilerParams(
        dimension_semantics=("parallel",),
        vmem_limit_bytes=96 << 20,
    )


def _run_stage_a(img3, params, H):
    N = img3.shape[0]
    wspecs, wargs = _const_specs(params)
    return pl.pallas_call(
        _stage_a_kernel,
        out_shape=jax.ShapeDtypeStruct((N // 2, H // 2, H // 2, 128),
                                       jnp.bfloat16),
        grid=(N // 2,),
        in_specs=[pl.BlockSpec((2, 3, H * H), lambda n: (n, 0, 0))] + wspecs,
        out_specs=pl.BlockSpec((1, H // 2, H // 2, 128),
                               lambda n: (n, 0, 0, 0)),
        compiler_params=_compiler_params(),
    )(img3, *wargs)


def _run_group(x, params, layers, batch, split_lanes=False, out_rows=None):
    G, H, W, _ = x.shape
    Ho, Wo = H, W
    for lay in layers:
        if lay['pool']:
            Ho, Wo = Ho // 2, Wo // 2
    Cout = params[-1][0].shape[-1]
    steps = G if split_lanes else G // batch
    n_out = out_rows if out_rows is not None else G

    wspecs, wargs = _const_specs(params)
    kern = functools.partial(_group_kernel, layers=layers, batch=batch,
                             split_lanes=split_lanes)
    in_rows = 1 if split_lanes else batch
    return pl.pallas_call(
        kern,
        out_shape=jax.ShapeDtypeStruct((n_out, Ho, Wo, Cout), jnp.bfloat16),
        grid=(steps,),
        in_specs=[pl.BlockSpec((in_rows,) + x.shape[1:],
                               lambda n: (n, 0, 0, 0))] + wspecs,
        out_specs=pl.BlockSpec((batch, Ho, Wo, Cout), lambda n: (n, 0, 0, 0)),
        compiler_params=_compiler_params(),
    )(x, *wargs)


def _pair_block_diag(w, single_dot):
    ci, co = w.shape[2], w.shape[3]
    eye = jnp.eye(2, dtype=w.dtype)
    wb = jnp.einsum('yxio,ab->yxaibo', w, eye).astype(jnp.bfloat16)
    if single_dot:
        return wb.reshape(9 * 2 * ci, 2 * co)
    return wb.reshape(3, 3 * 2 * ci, 2 * co)


def _pair_bias(b):
    return jnp.concatenate([b, b]).reshape(1, -1)


def kernel(img, w0, b0, w1, b1, w2, b2, w3, b3, w4, b4, w5, b5, w6, b6,
           w7, b7, w8, b8, w9, b9, w10, b10, w11, b11, w12, b12, w13, b13,
           w14, b14, w15, b15):
    ws = [w0, w1, w2, w3, w4, w5, w6, w7, w8, w9, w10, w11, w12, w13, w14, w15]
    bs = [b0, b1, b2, b3, b4, b5, b6, b7, b8, b9, b10, b11, b12, b13, b14, b15]
    N, _, H, _ = img.shape

    pa = [(jnp.transpose(ws[0].reshape(27, 64)), bs[0].reshape(64, 1)),
          (_pair_block_diag(ws[1], single_dot=False), _pair_bias(bs[1]))]
    x = _run_stage_a(img.reshape(N, 3, -1), pa, H)

    pb = [(_pair_block_diag(ws[2], single_dot=True), _pair_bias(bs[2])),
          (_pair_block_diag(ws[3], single_dot=True), _pair_bias(bs[3]))]
    x = _run_group(x, pb, _LAYERS[2:4], batch=1)

    def plain(i):
        lay = _LAYERS[i]
        if (3 * lay['cin']) % 128 == 0:
            return (ws[i].reshape(9 * lay['cin'], lay['cout']),
                    bs[i].reshape(1, lay['cout']))
        return (ws[i].reshape(3, 3 * lay['cin'], lay['cout']),
                bs[i].reshape(1, lay['cout']))

    x = _run_group(x, [plain(i) for i in range(4, 8)], _LAYERS[4:8],
                   batch=2, split_lanes=True, out_rows=N)
    x = _run_group(x, [plain(i) for i in range(8, 12)], _LAYERS[8:12], batch=2)
    x = _run_group(x, [plain(i) for i in range(12, 16)], _LAYERS[12:16],
                   batch=2)
    return jnp.transpose(x, (0, 3, 1, 2)).astype(jnp.float32)


# speedup vs baseline: 1.1535x; 1.1049x over previous
import functools

import jax
import jax.numpy as jnp
from jax.experimental import pallas as pl
from jax.experimental.pallas import tpu as pltpu


_CFG = [
    (64, True), (64, True), 'M',
    (128, True), (128, True), 'M',
    (256, True), (256, True), (256, True), (256, True), 'M',
    (512, True), (512, True), (512, True), (512, True), 'M',
    (512, True), (512, True), (512, True), (512, False),
]


def _layers():
    out, cin, i = [], 3, 0
    while i < len(_CFG):
        cout, relu = _CFG[i]
        pool = (i + 1 < len(_CFG)) and _CFG[i + 1] == 'M'
        out.append(dict(cin=cin, cout=cout, relu=relu, pool=pool))
        cin = cout
        i += 2 if pool else 1
    return out


_LAYERS = _layers()


def _first_conv_planar(x3, w27, b64):
    C, L = x3.shape
    W = int(round(L ** 0.5))
    lane = jax.lax.broadcasted_iota(jnp.int32, (C, L), 1) % W
    z1 = jnp.zeros((C, 1), jnp.bfloat16)
    sr = jnp.concatenate([z1, x3[:, :L - 1]], axis=1)
    sr = sr * (lane != 0).astype(jnp.bfloat16)
    sl = jnp.concatenate([x3[:, 1:], z1], axis=1)
    sl = sl * (lane != W - 1).astype(jnp.bfloat16)
    dx_rows = [sr, x3, sl]
    zW = jnp.zeros((C, W), jnp.bfloat16)
    rows = []
    for ky in range(3):
        for u in dx_rows:
            if ky == 0:
                rows.append(jnp.concatenate([zW, u[:, :L - W]], axis=1))
            elif ky == 2:
                rows.append(jnp.concatenate([u[:, W:], zW], axis=1))
            else:
                rows.append(u)
    taps = jnp.concatenate(rows, axis=0)
    acc = jnp.dot(w27, taps, preferred_element_type=jnp.float32)
    acc = jnp.maximum(acc + b64, 0.0)
    return acc.astype(jnp.bfloat16)


def _conv_bias_act(x, w3, b2, *, relu, pool):
    H, W, Cin = x.shape
    Cout = w3.shape[-1]

    zcol = jnp.zeros((H, 1, Cin), jnp.bfloat16)
    x_l = jnp.concatenate([zcol, x[:, :W - 1, :]], axis=1)
    x_r = jnp.concatenate([x[:, 1:, :], zcol], axis=1)
    xc = jnp.concatenate([x_l, x, x_r], axis=-1)
    zrow = jnp.zeros((1, W, 3 * Cin), jnp.bfloat16)
    xcp = jnp.concatenate([zrow, xc, zrow], axis=0)

    if w3.ndim == 2:
        xs = jnp.concatenate(
            [xcp[dy:dy + H] for dy in range(3)], axis=-1)
        acc = jnp.dot(xs.reshape(H * W, 9 * Cin), w3,
                      preferred_element_type=jnp.float32)
    else:
        acc = jnp.zeros((H * W, Cout), jnp.float32)
        for dy in range(3):
            xs = xcp[dy:dy + H].reshape(H * W, 3 * Cin)
            acc = acc + jnp.dot(xs, w3[dy],
                                preferred_element_type=jnp.float32)

    acc = acc + b2
    if relu:
        acc = jnp.maximum(acc, 0.0)

    if pool:
        ho, wo = H // 2, W // 2
        z = acc.reshape(H * wo, 2, Cout)
        z = jnp.maximum(z[:, 0, :], z[:, 1, :])
        z = z.reshape(ho, 2, wo, Cout)
        z = jnp.maximum(z[:, 0], z[:, 1])
        return z.astype(jnp.bfloat16)
    return acc.reshape(H, W, Cout).astype(jnp.bfloat16)


def _stage_a_kernel(x_ref, w27_ref, b1_ref, w2_ref, b2_ref, o_ref):
    y0 = _first_conv_planar(x_ref[0].astype(jnp.bfloat16),
                            w27_ref[...], b1_ref[...])
    y1 = _first_conv_planar(x_ref[1].astype(jnp.bfloat16),
                            w27_ref[...], b1_ref[...])
    yt = jnp.concatenate([y0, y1], axis=0)
    L = yt.shape[1]
    W = int(round(L ** 0.5))
    y = jnp.transpose(yt).reshape(L // W, W, 128)
    o_ref[0] = _conv_bias_act(y, w2_ref[...], b2_ref[...],
                              relu=True, pool=True)


def _im2col9(x):
    H, W, Cin = x.shape
    zcol = jnp.zeros((H, 1, Cin), jnp.bfloat16)
    x_l = jnp.concatenate([zcol, x[:, :W - 1, :]], axis=1)
    x_r = jnp.concatenate([x[:, 1:, :], zcol], axis=1)
    xc = jnp.concatenate([x_l, x, x_r], axis=-1)
    zrow = jnp.zeros((1, W, 3 * Cin), jnp.bfloat16)
    xcp = jnp.concatenate([zrow, xc, zrow], axis=0)
    xs = jnp.concatenate([xcp[dy:dy + H] for dy in range(3)], axis=-1)
    return xs.reshape(H * W, 9 * Cin)


def _epilogue(acc, b2, H, W, Cout, relu, pool):
    acc = acc + b2
    if relu:
        acc = jnp.maximum(acc, 0.0)
    if pool:
        ho, wo = H // 2, W // 2
        z = acc.reshape(H * wo, 2, Cout)
        z = jnp.maximum(z[:, 0, :], z[:, 1, :])
        z = z.reshape(ho, 2, wo, Cout)
        z = jnp.maximum(z[:, 0], z[:, 1])
        return z.astype(jnp.bfloat16)
    return acc.reshape(H, W, Cout).astype(jnp.bfloat16)


def _group_kernel(*refs, layers, batch, split_lanes):
    x_ref = refs[0]
    o_ref = refs[-1]
    if split_lanes:
        cin = layers[0]['cin']
        xs = [x_ref[0][..., b * cin:(b + 1) * cin] for b in range(batch)]
    else:
        xs = [x_ref[b] for b in range(batch)]
    for i, lay in enumerate(layers):
        w = refs[1 + 2 * i][...]
        b2 = refs[2 + 2 * i][...]
        H, W, _ = xs[0].shape
        Cout = w.shape[-1]
        if w.ndim == 2:
            pre = [_im2col9(x) for x in xs]
            accs = [jnp.dot(p, w, preferred_element_type=jnp.float32)
                    for p in pre]
            xs = [_epilogue(a, b2, H, W, Cout, lay['relu'], lay['pool'])
                  for a in accs]
        else:
            xs = [_conv_bias_act(x, w, b2, relu=lay['relu'],
                                 pool=lay['pool']) for x in xs]
    for b in range(batch):
        o_ref[b] = xs[b]


def _const_specs(params):
    specs, args = [], []
    for w, b in params:
        specs.append(pl.BlockSpec(w.shape, lambda n, _nd=w.ndim: (0,) * _nd))
        specs.append(pl.BlockSpec(b.shape, lambda n, _nd=b.ndim: (0,) * _nd))
        args.append(w)
        args.append(b)
    return specs, args


def _compiler_params():
    return pltpu.CompilerParams(
        dimension_semantics=("parallel",),
        vmem_limit_bytes=96 << 20,
    )


def _run_stage_a(img3, params, H):
    N = img3.shape[0]
    wspecs, wargs = _const_specs(params)
    return pl.pallas_call(
        _stage_a_kernel,
        out_shape=jax.ShapeDtypeStruct((N // 2, H // 2, H // 2, 128),
                                       jnp.bfloat16),
        grid=(N // 2,),
        in_specs=[pl.BlockSpec((2, 3, H * H), lambda n: (n, 0, 0))] + wspecs,
        out_specs=pl.BlockSpec((1, H // 2, H // 2, 128),
                               lambda n: (n, 0, 0, 0)),
        compiler_params=_compiler_params(),
    )(img3, *wargs)


def _run_group(x, params, layers, batch, split_lanes=False, out_rows=None):
    G, H, W, _ = x.shape
    Ho, Wo = H, W
    for lay in layers:
        if lay['pool']:
            Ho, Wo = Ho // 2, Wo // 2
    Cout = params[-1][0].shape[-1]
    steps = G if split_lanes else G // batch
    n_out = out_rows if out_rows is not None else G

    wspecs, wargs = _const_specs(params)
    kern = functools.partial(_group_kernel, layers=layers, batch=batch,
                             split_lanes=split_lanes)
    in_rows = 1 if split_lanes else batch
    return pl.pallas_call(
        kern,
        out_shape=jax.ShapeDtypeStruct((n_out, Ho, Wo, Cout), jnp.bfloat16),
        grid=(steps,),
        in_specs=[pl.BlockSpec((in_rows,) + x.shape[1:],
                               lambda n: (n, 0, 0, 0))] + wspecs,
        out_specs=pl.BlockSpec((batch, Ho, Wo, Cout), lambda n: (n, 0, 0, 0)),
        compiler_params=_compiler_params(),
    )(x, *wargs)


def _pair_block_diag(w, single_dot):
    ci, co = w.shape[2], w.shape[3]
    eye = jnp.eye(2, dtype=w.dtype)
    wb = jnp.einsum('yxio,ab->yxaibo', w, eye).astype(jnp.bfloat16)
    if single_dot:
        return wb.reshape(9 * 2 * ci, 2 * co)
    return wb.reshape(3, 3 * 2 * ci, 2 * co)


def _pair_bias(b):
    return jnp.concatenate([b, b]).reshape(1, -1)


def kernel(img, w0, b0, w1, b1, w2, b2, w3, b3, w4, b4, w5, b5, w6, b6,
           w7, b7, w8, b8, w9, b9, w10, b10, w11, b11, w12, b12, w13, b13,
           w14, b14, w15, b15):
    ws = [w0, w1, w2, w3, w4, w5, w6, w7, w8, w9, w10, w11, w12, w13, w14, w15]
    bs = [b0, b1, b2, b3, b4, b5, b6, b7, b8, b9, b10, b11, b12, b13, b14, b15]
    N, _, H, _ = img.shape

    pa = [(jnp.transpose(ws[0].reshape(27, 64)), bs[0].reshape(64, 1)),
          (_pair_block_diag(ws[1], single_dot=False), _pair_bias(bs[1]))]
    x = _run_stage_a(img.reshape(N, 3, -1), pa, H)

    pb = [(_pair_block_diag(ws[2], single_dot=True), _pair_bias(bs[2])),
          (_pair_block_diag(ws[3], single_dot=True), _pair_bias(bs[3]))]
    x = _run_group(x, pb, _LAYERS[2:4], batch=1)

    def plain(i):
        lay = _LAYERS[i]
        if (3 * lay['cin']) % 128 == 0:
            return (ws[i].reshape(9 * lay['cin'], lay['cout']),
                    bs[i].reshape(1, lay['cout']))
        return (ws[i].reshape(3, 3 * lay['cin'], lay['cout']),
                bs[i].reshape(1, lay['cout']))

    x = _run_group(x, [plain(i) for i in range(4, 8)], _LAYERS[4:8],
                   batch=2, split_lanes=True, out_rows=N)
    x = _run_group(x, [plain(i) for i in range(8, 12)], _LAYERS[8:12], batch=2)
    x = _run_group(x, [plain(i) for i in range(12, 16)], _LAYERS[12:16],
                   batch=2)
    return jnp.transpose(x, (0, 3, 1, 2)).astype(jnp.float32)


# speedup vs baseline: 1.1571x; 1.0031x over previous
import functools

import jax
import jax.numpy as jnp
from jax.experimental import pallas as pl
from jax.experimental.pallas import tpu as pltpu


_CFG = [
    (64, True), (64, True), 'M',
    (128, True), (128, True), 'M',
    (256, True), (256, True), (256, True), (256, True), 'M',
    (512, True), (512, True), (512, True), (512, True), 'M',
    (512, True), (512, True), (512, True), (512, False),
]


def _layers():
    out, cin, i = [], 3, 0
    while i < len(_CFG):
        cout, relu = _CFG[i]
        pool = (i + 1 < len(_CFG)) and _CFG[i + 1] == 'M'
        out.append(dict(cin=cin, cout=cout, relu=relu, pool=pool))
        cin = cout
        i += 2 if pool else 1
    return out


_LAYERS = _layers()


def _first_conv_planar(x3, w27, b64):
    C, L = x3.shape
    W = int(round(L ** 0.5))
    lane = jax.lax.broadcasted_iota(jnp.int32, (C, L), 1) % W
    z1 = jnp.zeros((C, 1), jnp.bfloat16)
    sr = jnp.concatenate([z1, x3[:, :L - 1]], axis=1)
    sr = sr * (lane != 0).astype(jnp.bfloat16)
    sl = jnp.concatenate([x3[:, 1:], z1], axis=1)
    sl = sl * (lane != W - 1).astype(jnp.bfloat16)
    dx_rows = [sr, x3, sl]
    zW = jnp.zeros((C, W), jnp.bfloat16)
    rows = []
    for ky in range(3):
        for u in dx_rows:
            if ky == 0:
                rows.append(jnp.concatenate([zW, u[:, :L - W]], axis=1))
            elif ky == 2:
                rows.append(jnp.concatenate([u[:, W:], zW], axis=1))
            else:
                rows.append(u)
    taps = jnp.concatenate(rows, axis=0)
    acc = jnp.dot(w27, taps, preferred_element_type=jnp.float32)
    acc = jnp.maximum(acc + b64, 0.0)
    return acc.astype(jnp.bfloat16)


def _conv_bias_act(x, w3, b2, *, relu, pool):
    H, W, Cin = x.shape
    Cout = w3.shape[-1]

    zcol = jnp.zeros((H, 1, Cin), jnp.bfloat16)
    x_l = jnp.concatenate([zcol, x[:, :W - 1, :]], axis=1)
    x_r = jnp.concatenate([x[:, 1:, :], zcol], axis=1)
    xc = jnp.concatenate([x_l, x, x_r], axis=-1)
    zrow = jnp.zeros((1, W, 3 * Cin), jnp.bfloat16)
    xcp = jnp.concatenate([zrow, xc, zrow], axis=0)

    if w3.ndim == 2:
        xs = jnp.concatenate(
            [xcp[dy:dy + H] for dy in range(3)], axis=-1)
        acc = jnp.dot(xs.reshape(H * W, 9 * Cin), w3,
                      preferred_element_type=jnp.float32)
    else:
        acc = jnp.zeros((H * W, Cout), jnp.float32)
        for dy in range(3):
            xs = xcp[dy:dy + H].reshape(H * W, 3 * Cin)
            acc = acc + jnp.dot(xs, w3[dy],
                                preferred_element_type=jnp.float32)

    acc = acc + b2
    if relu:
        acc = jnp.maximum(acc, 0.0)

    if pool:
        ho, wo = H // 2, W // 2
        z = acc.reshape(H * wo, 2, Cout)
        z = jnp.maximum(z[:, 0, :], z[:, 1, :])
        z = z.reshape(ho, 2, wo, Cout)
        z = jnp.maximum(z[:, 0], z[:, 1])
        return z.astype(jnp.bfloat16)
    return acc.reshape(H, W, Cout).astype(jnp.bfloat16)


def _stage_a_kernel(x_ref, w27_ref, b1_ref, w2_ref, b2_ref, o_ref):
    y0 = _first_conv_planar(x_ref[0].astype(jnp.bfloat16),
                            w27_ref[...], b1_ref[...])
    y1 = _first_conv_planar(x_ref[1].astype(jnp.bfloat16),
                            w27_ref[...], b1_ref[...])
    yt = jnp.concatenate([y0, y1], axis=0)
    L = yt.shape[1]
    W = int(round(L ** 0.5))
    y = jnp.transpose(yt).reshape(L // W, W, 128)
    o_ref[0] = _conv_bias_act(y, w2_ref[...], b2_ref[...],
                              relu=True, pool=True)


def _im2col9(x):
    H, W, Cin = x.shape
    zcol = jnp.zeros((H, 1, Cin), jnp.bfloat16)
    x_l = jnp.concatenate([zcol, x[:, :W - 1, :]], axis=1)
    x_r = jnp.concatenate([x[:, 1:, :], zcol], axis=1)
    xc = jnp.concatenate([x_l, x, x_r], axis=-1)
    zrow = jnp.zeros((1, W, 3 * Cin), jnp.bfloat16)
    xcp = jnp.concatenate([zrow, xc, zrow], axis=0)
    xs = jnp.concatenate([xcp[dy:dy + H] for dy in range(3)], axis=-1)
    return xs.reshape(H * W, 9 * Cin)


def _epilogue(acc, b2, H, W, Cout, relu, pool):
    acc = acc + b2
    if relu:
        acc = jnp.maximum(acc, 0.0)
    if pool:
        ho, wo = H // 2, W // 2
        z = acc.reshape(H * wo, 2, Cout)
        z = jnp.maximum(z[:, 0, :], z[:, 1, :])
        z = z.reshape(ho, 2, wo, Cout)
        z = jnp.maximum(z[:, 0], z[:, 1])
        return z.astype(jnp.bfloat16)
    return acc.reshape(H, W, Cout).astype(jnp.bfloat16)


def _group_kernel(*refs, layers, batch, split_lanes):
    x_ref = refs[0]
    o_ref = refs[-1]
    if split_lanes:
        cin = layers[0]['cin']
        xs = [x_ref[0][..., b * cin:(b + 1) * cin] for b in range(batch)]
    else:
        xs = [x_ref[b] for b in range(batch)]
    for i, lay in enumerate(layers):
        w = refs[1 + 2 * i][...]
        b2 = refs[2 + 2 * i][...]
        H, W, _ = xs[0].shape
        Cout = w.shape[-1]
        if w.ndim == 2:
            pre = [_im2col9(x) for x in xs]
            accs = [jnp.dot(p, w, preferred_element_type=jnp.float32)
                    for p in pre]
            xs = [_epilogue(a, b2, H, W, Cout, lay['relu'], lay['pool'])
                  for a in accs]
        else:
            xs = [_conv_bias_act(x, w, b2, relu=lay['relu'],
                                 pool=lay['pool']) for x in xs]
    for b in range(batch):
        o_ref[b] = xs[b]


def _const_specs(params):
    specs, args = [], []
    for w, b in params:
        specs.append(pl.BlockSpec(w.shape, lambda n, _nd=w.ndim: (0,) * _nd))
        specs.append(pl.BlockSpec(b.shape, lambda n, _nd=b.ndim: (0,) * _nd))
        args.append(w)
        args.append(b)
    return specs, args


def _compiler_params():
    return pltpu.CompilerParams(
        dimension_semantics=("parallel",),
        vmem_limit_bytes=96 << 20,
    )


def _run_stage_a(img3, params, H):
    N = img3.shape[0]
    wspecs, wargs = _const_specs(params)
    return pl.pallas_call(
        _stage_a_kernel,
        out_shape=jax.ShapeDtypeStruct((N // 2, H // 2, H // 2, 128),
                                       jnp.bfloat16),
        grid=(N // 2,),
        in_specs=[pl.BlockSpec((2, 3, H * H), lambda n: (n, 0, 0))] + wspecs,
        out_specs=pl.BlockSpec((1, H // 2, H // 2, 128),
                               lambda n: (n, 0, 0, 0)),
        compiler_params=_compiler_params(),
    )(img3, *wargs)


def _run_group(x, params, layers, batch, split_lanes=False, out_rows=None):
    G, H, W, _ = x.shape
    Ho, Wo = H, W
    for lay in layers:
        if lay['pool']:
            Ho, Wo = Ho // 2, Wo // 2
    Cout = params[-1][0].shape[-1]
    steps = G if split_lanes else G // batch
    n_out = out_rows if out_rows is not None else G

    wspecs, wargs = _const_specs(params)
    kern = functools.partial(_group_kernel, layers=layers, batch=batch,
                             split_lanes=split_lanes)
    in_rows = 1 if split_lanes else batch
    return pl.pallas_call(
        kern,
        out_shape=jax.ShapeDtypeStruct((n_out, Ho, Wo, Cout), jnp.bfloat16),
        grid=(steps,),
        in_specs=[pl.BlockSpec((in_rows,) + x.shape[1:],
                               lambda n: (n, 0, 0, 0))] + wspecs,
        out_specs=pl.BlockSpec((batch, Ho, Wo, Cout), lambda n: (n, 0, 0, 0)),
        compiler_params=_compiler_params(),
    )(x, *wargs)


def _pair_block_diag(w, single_dot):
    ci, co = w.shape[2], w.shape[3]
    eye = jnp.eye(2, dtype=w.dtype)
    wb = jnp.einsum('yxio,ab->yxaibo', w, eye).astype(jnp.bfloat16)
    if single_dot:
        return wb.reshape(9 * 2 * ci, 2 * co)
    return wb.reshape(3, 3 * 2 * ci, 2 * co)


def _pair_bias(b):
    return jnp.concatenate([b, b]).reshape(1, -1)


def kernel(img, w0, b0, w1, b1, w2, b2, w3, b3, w4, b4, w5, b5, w6, b6,
           w7, b7, w8, b8, w9, b9, w10, b10, w11, b11, w12, b12, w13, b13,
           w14, b14, w15, b15):
    ws = [w0, w1, w2, w3, w4, w5, w6, w7, w8, w9, w10, w11, w12, w13, w14, w15]
    bs = [b0, b1, b2, b3, b4, b5, b6, b7, b8, b9, b10, b11, b12, b13, b14, b15]
    N, _, H, _ = img.shape

    pa = [(jnp.transpose(ws[0].reshape(27, 64)), bs[0].reshape(64, 1)),
          (_pair_block_diag(ws[1], single_dot=False), _pair_bias(bs[1]))]
    x = _run_stage_a(img.reshape(N, 3, -1), pa, H)

    pb = [(_pair_block_diag(ws[2], single_dot=True), _pair_bias(bs[2])),
          (_pair_block_diag(ws[3], single_dot=True), _pair_bias(bs[3]))]
    x = _run_group(x, pb, _LAYERS[2:4], batch=1)

    def plain(i):
        lay = _LAYERS[i]
        if (3 * lay['cin']) % 128 == 0:
            return (ws[i].reshape(9 * lay['cin'], lay['cout']),
                    bs[i].reshape(1, lay['cout']))
        return (ws[i].reshape(3, 3 * lay['cin'], lay['cout']),
                bs[i].reshape(1, lay['cout']))

    x = _run_group(x, [plain(i) for i in range(4, 8)], _LAYERS[4:8],
                   batch=2, split_lanes=True, out_rows=N)
    x = _run_group(x, [plain(i) for i in range(8, 12)], _LAYERS[8:12], batch=4)
    x = _run_group(x, [plain(i) for i in range(12, 16)], _LAYERS[12:16],
                   batch=4)
    return jnp.transpose(x, (0, 3, 1, 2)).astype(jnp.float32)
